# Initial kernel scaffold; baseline (speedup 1.0000x reference)
import jax
import jax.numpy as jnp
from jax.experimental import pallas as pl


def kernel(x, edge_index, edge_attr, event1, event2, c0_W, c0_b, c0_Ws, c0_bs, c1_W, c1_b, c1_Ws, c1_bs, ln_g, ln_b, lin_W, lin_b, mp1_W, mp1_b, mp2_W, mp2_b):
    raise NotImplementedError("write your pallas kernel here")


# speedup vs baseline: 1.7839x; 1.7839x over previous
import jax
import jax.numpy as jnp
from jax import lax
from jax.experimental import pallas as pl
from jax.experimental.pallas import tpu as pltpu
from jax.experimental.pallas import tpu_sc as plsc

_N = 10000
_E = 320000
_D = 128
_H = 50
_HP = 64
_F = 4
_P = 1024
_R = 3

_NC = 2
_NS = 16
_NW = _NC * _NS
_EPW = _E // _NW
_BE = 80
_NBLK = _EPW // _BE
_NP = 10240
_ROWS = _NP // _NS
_HW = 128
_EVT = 2 * _P
_EVB = _EVT // _NS

_ROWBLK = 400


def _edge_compute(attr_v, rows_v, msg_v):
    def grp(g, carry):
        av = [attr_v[f, pl.ds(g * 16, 16)] for f in range(_F)]
        for j in range(16):
            e = g * 16 + j
            jv = jnp.full((16,), j, dtype=jnp.int32)
            bf = [
                jnp.take_along_axis(av[f], jv, axis=0,
                                    mode="promise_in_bounds")
                for f in range(_F)
            ]
            for cc in range(_HP // 16):
                m = bf[0] * rows_v[e, pl.ds(cc * 16, 16)]
                for f in range(1, _F):
                    m = m + bf[f] * rows_v[e, pl.ds(f * _HP + cc * 16, 16)]
                msg_v[e, pl.ds(cc * 16, 16)] = m
        return carry

    lax.fori_loop(0, _BE // 16, grp, 0)


def _seed_and_agg(yhat, src, dst, attrF, seed,
                  idx_v, didx_v, attr_v, rows_v, msg_v, accum, sem, c, s):
    wid = c * _NS + s
    stripe = pl.ds(s * _ROWS, _ROWS)

    zv = jnp.zeros((16,), jnp.float32)

    def zmsg(e, carry):
        for cc in range(_HP // 16):
            msg_v[e, pl.ds(_HP + cc * 16, 16)] = zv
        return carry

    lax.fori_loop(0, _BE, zmsg, 0)
    pltpu.sync_copy(seed.at[c, stripe], accum.at[stripe])
    plsc.subcore_barrier()

    def blk(i, carry):
        base = wid * _EPW + i * _BE
        pltpu.sync_copy(src.at[pl.ds(base, _BE)], idx_v)
        pltpu.sync_copy(dst.at[pl.ds(base, _BE)], didx_v)
        for f in range(_F):
            pltpu.sync_copy(attrF.at[pl.ds(f * _E + base, _BE)], attr_v.at[f])
        pltpu.async_copy(yhat.at[idx_v], rows_v, sem).wait()
        _edge_compute(attr_v, rows_v, msg_v)
        pltpu.sync_copy(msg_v, accum.at[didx_v], add=True)
        return carry

    lax.fori_loop(0, _NBLK, blk, 0)
    plsc.subcore_barrier()


def _sc_agg_full_body(yhat, src, dst, attrF, seed, out,
                      idx_v, didx_v, attr_v, rows_v, msg_v, accum, sem):
    c = lax.axis_index("c")
    s = lax.axis_index("s")
    _seed_and_agg(yhat, src, dst, attrF, seed,
                  idx_v, didx_v, attr_v, rows_v, msg_v, accum, sem, c, s)
    stripe = pl.ds(s * _ROWS, _ROWS)
    pltpu.sync_copy(accum.at[stripe], out.at[c, stripe])


def _sc_agg_ev_body(yhat, src, dst, attrF, seed, evcat, out,
                    idx_v, didx_v, attr_v, rows_v, msg_v,
                    evidx_v, evrows_v, accum, sem):
    c = lax.axis_index("c")
    s = lax.axis_index("s")
    _seed_and_agg(yhat, src, dst, attrF, seed,
                  idx_v, didx_v, attr_v, rows_v, msg_v, accum, sem, c, s)
    evb = s * _EVB
    pltpu.sync_copy(evcat.at[pl.ds(evb, _EVB)], evidx_v)
    pltpu.async_copy(accum.at[evidx_v], evrows_v, sem).wait()
    pltpu.sync_copy(evrows_v, out.at[c, pl.ds(evb, _EVB)])


def _sc_mesh():
    return plsc.VectorSubcoreMesh(core_axis_name="c", subcore_axis_name="s")


_SC_COMMON_SCRATCH = [
    pltpu.VMEM((_BE,), jnp.int32),
    pltpu.VMEM((_BE,), jnp.int32),
    pltpu.VMEM((_F, _BE), jnp.float32),
    pltpu.VMEM((_BE, _F * _HP), jnp.float32),
    pltpu.VMEM((_BE, _HW), jnp.float32),
]


def _sc_agg_full(yhat, src, dst, attrF, seed):
    return pl.kernel(
        _sc_agg_full_body,
        out_type=jax.ShapeDtypeStruct((_NC, _NP, _HW), jnp.float32),
        mesh=_sc_mesh(),
        scratch_types=_SC_COMMON_SCRATCH + [
            pltpu.VMEM_SHARED((_NP, _HW), jnp.float32),
            pltpu.SemaphoreType.DMA,
        ],
    )(yhat, src, dst, attrF, seed)


def _sc_agg_ev(yhat, src, dst, attrF, seed, evcat):
    return pl.kernel(
        _sc_agg_ev_body,
        out_type=jax.ShapeDtypeStruct((_NC, _EVT, _HW), jnp.float32),
        mesh=_sc_mesh(),
        scratch_types=_SC_COMMON_SCRATCH + [
            pltpu.VMEM((_EVB,), jnp.int32),
            pltpu.VMEM((_EVB, _HW), jnp.float32),
            pltpu.VMEM_SHARED((_NP, _HW), jnp.float32),
            pltpu.SemaphoreType.DMA,
        ],
    )(yhat, src, dst, attrF, seed, evcat)


def _tca_body(x_ref, w_ref, b_ref, yhat_ref, self_ref):
    t = jnp.dot(x_ref[...], w_ref[...], preferred_element_type=jnp.float32)
    t = t + b_ref[...]
    yhat_ref[...] = t[:, : _F * _HP]
    sf = t[:, _F * _HP:]
    self_ref[0] = jnp.concatenate([sf, jnp.zeros_like(sf)], axis=1)
    self_ref[1] = jnp.zeros((sf.shape[0], _HW), jnp.float32)


def _tc_linear(x, w, b):
    din = x.shape[1]
    return pl.pallas_call(
        _tca_body,
        grid=(_N // _ROWBLK,),
        in_specs=[
            pl.BlockSpec((_ROWBLK, din), lambda i: (i, 0)),
            pl.BlockSpec(w.shape, lambda i: (0, 0)),
            pl.BlockSpec(b.shape, lambda i: (0, 0)),
        ],
        out_specs=[
            pl.BlockSpec((_ROWBLK, _F * _HP), lambda i: (i, 0)),
            pl.BlockSpec((_NC, _ROWBLK, _HW), lambda i: (0, i, 0)),
        ],
        out_shape=[
            jax.ShapeDtypeStruct((_N, _F * _HP), jnp.float32),
            jax.ShapeDtypeStruct((_NC, _NP, _HW), jnp.float32),
        ],
    )(x, w, b)


def _tcb_body(p0_ref, p1_ref, g_ref, bb_ref, w_ref, b_ref,
              yhat_ref, self_ref):
    t = (p0_ref[0] + p1_ref[0])[:, :_HP]
    mean = jnp.sum(t, axis=-1, keepdims=True) * (1.0 / _H)
    d = t - mean
    lane = lax.broadcasted_iota(jnp.int32, t.shape, 1)
    d = jnp.where(lane < _H, d, 0.0)
    var = jnp.sum(d * d, axis=-1, keepdims=True) * (1.0 / _H)
    h1 = d * lax.rsqrt(var + 1e-5) * g_ref[...] + bb_ref[...]
    t2 = jnp.dot(h1, w_ref[...], preferred_element_type=jnp.float32)
    t2 = t2 + b_ref[...]
    yhat_ref[...] = t2[:, : _F * _HP]
    sf = t2[:, _F * _HP:]
    self_ref[0] = jnp.concatenate([sf, jnp.zeros_like(sf)], axis=1)
    self_ref[1] = jnp.zeros((sf.shape[0], _HW), jnp.float32)


def _tc_ln_linear(parts, g, bb, w, b):
    return pl.pallas_call(
        _tcb_body,
        grid=(_N // _ROWBLK,),
        in_specs=[
            pl.BlockSpec((1, _ROWBLK, _HW), lambda i: (0, i, 0)),
            pl.BlockSpec((1, _ROWBLK, _HW), lambda i: (1, i, 0)),
            pl.BlockSpec(g.shape, lambda i: (0, 0)),
            pl.BlockSpec(bb.shape, lambda i: (0, 0)),
            pl.BlockSpec(w.shape, lambda i: (0, 0)),
            pl.BlockSpec(b.shape, lambda i: (0, 0)),
        ],
        out_specs=[
            pl.BlockSpec((_ROWBLK, _F * _HP), lambda i: (i, 0)),
            pl.BlockSpec((_NC, _ROWBLK, _HW), lambda i: (0, i, 0)),
        ],
        out_shape=[
            jax.ShapeDtypeStruct((_N, _F * _HP), jnp.float32),
            jax.ShapeDtypeStruct((_NC, _NP, _HW), jnp.float32),
        ],
    )(parts, parts, g, bb, w, b)


def _tcd_body(ev_ref, lw_ref, lb_ref, m1a_ref, m1b_ref, m1bias_ref,
              m2_ref, m2b_ref, o_ref):
    ht = (ev_ref[0] + ev_ref[1])[:, :_HP]
    h2 = jnp.dot(ht, lw_ref[...], preferred_element_type=jnp.float32)
    h2 = h2 + lb_ref[...]
    ea = h2[:_P]
    eb = h2[_P:]
    z = (jnp.dot(ea, m1a_ref[...], preferred_element_type=jnp.float32)
         + jnp.dot(eb, m1b_ref[...], preferred_element_type=jnp.float32)
         + m1bias_ref[...])
    z = jnp.where(z >= 0.0, z, 0.01 * z)
    o_ref[...] = jnp.dot(z, m2_ref[...], preferred_element_type=jnp.float32) + m2b_ref[...]


def _tc_head(evparts, lw, lb, m1a, m1b_w, m1bias, m2, m2bias):
    return pl.pallas_call(
        _tcd_body,
        out_shape=jax.ShapeDtypeStruct((_P, 128), jnp.float32),
    )(evparts, lw, lb, m1a, m1b_w, m1bias, m2, m2bias)


def _pack_conv_w(W, b, Ws, bs, din_pad):
    din = W.shape[2]
    wt = jnp.transpose(W, (2, 0, 1))
    wt = jnp.pad(wt, ((0, din_pad - din), (0, 0), (0, _HP - _H)))
    wt = wt.reshape(din_pad, _F * _HP)
    ws = jnp.pad(jnp.transpose(Ws), ((0, din_pad - din), (0, _HP - _H)))
    wall = jnp.concatenate([wt, ws], axis=1)
    bt = jnp.pad(b, ((0, 0), (0, _HP - _H))).reshape(_F * _HP)
    bsp = jnp.pad(bs, (0, _HP - _H))
    ball = jnp.concatenate([bt, bsp])[None, :]
    return wall, ball


def kernel(x, edge_index, edge_attr, event1, event2, c0_W, c0_b, c0_Ws, c0_bs,
           c1_W, c1_b, c1_Ws, c1_bs, ln_g, ln_b, lin_W, lin_b, mp1_W, mp1_b,
           mp2_W, mp2_b):
    src = edge_index[0]
    dst = edge_index[1]
    attrF = jnp.transpose(edge_attr).reshape(-1)
    evcat = jnp.concatenate([event1, event2])

    w0, b0 = _pack_conv_w(c0_W, c0_b, c0_Ws, c0_bs, _D)
    w1, b1 = _pack_conv_w(c1_W, c1_b, c1_Ws, c1_bs, _HP)
    gpad = jnp.pad(ln_g, (0, _HP - _H))[None, :]
    bpad = jnp.pad(ln_b, (0, _HP - _H))[None, :]
    lwp = jnp.pad(jnp.transpose(lin_W), ((0, _HP - _H), (0, _HP - _H)))
    lbp = jnp.pad(lin_b, (0, _HP - _H))[None, :]
    m1a = jnp.pad(jnp.transpose(mp1_W[:, :_H]), ((0, _HP - _H), (0, _HP - _H)))
    m1b_w = jnp.pad(jnp.transpose(mp1_W[:, _H:]), ((0, _HP - _H), (0, _HP - _H)))
    m1bias = jnp.pad(mp1_b, (0, _HP - _H))[None, :]
    m2 = jnp.pad(jnp.transpose(mp2_W), ((0, _HP - _H), (0, 128 - _R)))
    m2bias = jnp.pad(mp2_b, (0, 128 - _R))[None, :]

    yhat0, seed0 = _tc_linear(x, w0, b0)
    parts0 = _sc_agg_full(yhat0, src, dst, attrF, seed0)
    yhat1, seed1 = _tc_ln_linear(parts0, gpad, bpad, w1, b1)
    evparts = _sc_agg_ev(yhat1, src, dst, attrF, seed1, evcat)
    zfull = _tc_head(evparts, lwp, lbp, m1a, m1b_w, m1bias, m2, m2bias)
    return zfull[:, :_R]


# speedup vs baseline: 2.2179x; 1.2432x over previous
import jax
import jax.numpy as jnp
from jax import lax
from jax.experimental import pallas as pl
from jax.experimental.pallas import tpu as pltpu
from jax.experimental.pallas import tpu_sc as plsc

_N = 10000
_E = 320000
_D = 128
_H = 50
_HP = 64
_F = 4
_P = 1024
_R = 3

_NC = 2
_NS = 16
_NW = _NC * _NS
_EPW = _E // _NW
_BE = 80
_NBLK = _EPW // _BE
_NP = 10240
_ROWS = _NP // _NS
_HW = 128
_EVT = 2 * _P
_EVB = _EVT // _NS

_ROWBLK = 400


def _edge_compute(attr_v, rows_v, msg_v):
    def grp(g, carry):
        av = [attr_v[pl.ds(f * _BE + g * 16, 16)] for f in range(_F)]
        for j in range(16):
            e = g * 16 + j
            jv = jnp.full((16,), j, dtype=jnp.int32)
            bf = [
                jnp.take_along_axis(av[f], jv, axis=0,
                                    mode="promise_in_bounds")
                for f in range(_F)
            ]
            for cc in range(_HP // 16):
                m = bf[0] * rows_v[e, pl.ds(cc * 16, 16)]
                for f in range(1, _F):
                    m = m + bf[f] * rows_v[e, pl.ds(f * _HP + cc * 16, 16)]
                msg_v[e, pl.ds(cc * 16, 16)] = m
        return carry

    lax.fori_loop(0, _BE // 16, grp, 0)


def _seed_and_agg(yhat, src, dst, attrF, seed,
                  idx_v, didx_v, attr_v, rows_v, msg_v, accum, sem, sem2,
                  c, s):
    wid = c * _NS + s
    stripe = pl.ds(s * _ROWS, _ROWS)

    zv = jnp.zeros((16,), jnp.float32)

    def zmsg(e, carry):
        for cc in range(_HP // 16):
            msg_v[e, pl.ds(_HP + cc * 16, 16)] = zv
        return carry

    lax.fori_loop(0, _BE, zmsg, 0)
    pltpu.sync_copy(seed.at[c, stripe], accum.at[stripe])
    plsc.subcore_barrier()

    def blk(i, carry):
        base = wid * _EPW + i * _BE
        pltpu.sync_copy(src.at[pl.ds(base, _BE)], idx_v)
        pltpu.sync_copy(dst.at[pl.ds(base, _BE)], didx_v)
        pltpu.sync_copy(attrF.at[pl.ds(base * _F, _F * _BE)], attr_v)
        pltpu.async_copy(yhat.at[idx_v], rows_v, sem).wait()

        @pl.when(i > 0)
        def _():
            pltpu.make_async_copy(msg_v, accum.at[didx_v], sem2).wait()

        _edge_compute(attr_v, rows_v, msg_v)
        pltpu.async_copy(msg_v, accum.at[didx_v], sem2, add=True)
        return carry

    lax.fori_loop(0, _NBLK, blk, 0)
    pltpu.make_async_copy(msg_v, accum.at[didx_v], sem2).wait()
    plsc.subcore_barrier()


def _sc_agg_full_body(yhat, src, dst, attrF, seed, out,
                      idx_v, didx_v, attr_v, rows_v, msg_v, accum, sem, sem2):
    c = lax.axis_index("c")
    s = lax.axis_index("s")
    _seed_and_agg(yhat, src, dst, attrF, seed,
                  idx_v, didx_v, attr_v, rows_v, msg_v, accum, sem, sem2, c, s)
    stripe = pl.ds(s * _ROWS, _ROWS)
    pltpu.sync_copy(accum.at[stripe], out.at[c, stripe])


def _sc_agg_ev_body(yhat, src, dst, attrF, seed, evcat, out,
                    idx_v, didx_v, attr_v, rows_v, msg_v,
                    evidx_v, evrows_v, accum, sem, sem2):
    c = lax.axis_index("c")
    s = lax.axis_index("s")
    _seed_and_agg(yhat, src, dst, attrF, seed,
                  idx_v, didx_v, attr_v, rows_v, msg_v, accum, sem, sem2, c, s)
    evb = s * _EVB
    pltpu.sync_copy(evcat.at[pl.ds(evb, _EVB)], evidx_v)
    pltpu.async_copy(accum.at[evidx_v], evrows_v, sem).wait()
    pltpu.sync_copy(evrows_v, out.at[c, pl.ds(evb, _EVB)])


def _sc_mesh():
    return plsc.VectorSubcoreMesh(core_axis_name="c", subcore_axis_name="s")


_SC_COMMON_SCRATCH = [
    pltpu.VMEM((_BE,), jnp.int32),
    pltpu.VMEM((_BE,), jnp.int32),
    pltpu.VMEM((_F * _BE,), jnp.float32),
    pltpu.VMEM((_BE, _F * _HP), jnp.float32),
    pltpu.VMEM((_BE, _HW), jnp.float32),
]


def _sc_agg_full(yhat, src, dst, attrF, seed):
    return pl.kernel(
        _sc_agg_full_body,
        out_type=jax.ShapeDtypeStruct((_NC, _NP, _HW), jnp.float32),
        mesh=_sc_mesh(),
        scratch_types=_SC_COMMON_SCRATCH + [
            pltpu.VMEM_SHARED((_NP, _HW), jnp.float32),
            pltpu.SemaphoreType.DMA,
            pltpu.SemaphoreType.DMA,
        ],
    )(yhat, src, dst, attrF, seed)


def _sc_agg_ev(yhat, src, dst, attrF, seed, evcat):
    return pl.kernel(
        _sc_agg_ev_body,
        out_type=jax.ShapeDtypeStruct((_NC, _EVT, _HW), jnp.float32),
        mesh=_sc_mesh(),
        scratch_types=_SC_COMMON_SCRATCH + [
            pltpu.VMEM((_EVB,), jnp.int32),
            pltpu.VMEM((_EVB, _HW), jnp.float32),
            pltpu.VMEM_SHARED((_NP, _HW), jnp.float32),
            pltpu.SemaphoreType.DMA,
            pltpu.SemaphoreType.DMA,
        ],
    )(yhat, src, dst, attrF, seed, evcat)


def _tca_body(x_ref, w_ref, b_ref, yhat_ref, self_ref):
    t = jnp.dot(x_ref[...], w_ref[...], preferred_element_type=jnp.float32)
    t = t + b_ref[...]
    yhat_ref[...] = t[:, : _F * _HP]
    sf = t[:, _F * _HP:]
    self_ref[0] = jnp.concatenate([sf, jnp.zeros_like(sf)], axis=1)
    self_ref[1] = jnp.zeros((sf.shape[0], _HW), jnp.float32)


def _tc_linear(x, w, b):
    din = x.shape[1]
    return pl.pallas_call(
        _tca_body,
        grid=(_N // _ROWBLK,),
        in_specs=[
            pl.BlockSpec((_ROWBLK, din), lambda i: (i, 0)),
            pl.BlockSpec(w.shape, lambda i: (0, 0)),
            pl.BlockSpec(b.shape, lambda i: (0, 0)),
        ],
        out_specs=[
            pl.BlockSpec((_ROWBLK, _F * _HP), lambda i: (i, 0)),
            pl.BlockSpec((_NC, _ROWBLK, _HW), lambda i: (0, i, 0)),
        ],
        out_shape=[
            jax.ShapeDtypeStruct((_N, _F * _HP), jnp.float32),
            jax.ShapeDtypeStruct((_NC, _NP, _HW), jnp.float32),
        ],
    )(x, w, b)


def _tcb_body(p0_ref, p1_ref, g_ref, bb_ref, w_ref, b_ref,
              yhat_ref, self_ref):
    t = (p0_ref[0] + p1_ref[0])[:, :_HP]
    mean = jnp.sum(t, axis=-1, keepdims=True) * (1.0 / _H)
    d = t - mean
    lane = lax.broadcasted_iota(jnp.int32, t.shape, 1)
    d = jnp.where(lane < _H, d, 0.0)
    var = jnp.sum(d * d, axis=-1, keepdims=True) * (1.0 / _H)
    h1 = d * lax.rsqrt(var + 1e-5) * g_ref[...] + bb_ref[...]
    t2 = jnp.dot(h1, w_ref[...], preferred_element_type=jnp.float32)
    t2 = t2 + b_ref[...]
    yhat_ref[...] = t2[:, : _F * _HP]
    sf = t2[:, _F * _HP:]
    self_ref[0] = jnp.concatenate([sf, jnp.zeros_like(sf)], axis=1)
    self_ref[1] = jnp.zeros((sf.shape[0], _HW), jnp.float32)


def _tc_ln_linear(parts, g, bb, w, b):
    return pl.pallas_call(
        _tcb_body,
        grid=(_N // _ROWBLK,),
        in_specs=[
            pl.BlockSpec((1, _ROWBLK, _HW), lambda i: (0, i, 0)),
            pl.BlockSpec((1, _ROWBLK, _HW), lambda i: (1, i, 0)),
            pl.BlockSpec(g.shape, lambda i: (0, 0)),
            pl.BlockSpec(bb.shape, lambda i: (0, 0)),
            pl.BlockSpec(w.shape, lambda i: (0, 0)),
            pl.BlockSpec(b.shape, lambda i: (0, 0)),
        ],
        out_specs=[
            pl.BlockSpec((_ROWBLK, _F * _HP), lambda i: (i, 0)),
            pl.BlockSpec((_NC, _ROWBLK, _HW), lambda i: (0, i, 0)),
        ],
        out_shape=[
            jax.ShapeDtypeStruct((_N, _F * _HP), jnp.float32),
            jax.ShapeDtypeStruct((_NC, _NP, _HW), jnp.float32),
        ],
    )(parts, parts, g, bb, w, b)


def _tcd_body(ev_ref, lw_ref, lb_ref, m1a_ref, m1b_ref, m1bias_ref,
              m2_ref, m2b_ref, o_ref):
    ht = (ev_ref[0] + ev_ref[1])[:, :_HP]
    h2 = jnp.dot(ht, lw_ref[...], preferred_element_type=jnp.float32)
    h2 = h2 + lb_ref[...]
    ea = h2[:_P]
    eb = h2[_P:]
    z = (jnp.dot(ea, m1a_ref[...], preferred_element_type=jnp.float32)
         + jnp.dot(eb, m1b_ref[...], preferred_element_type=jnp.float32)
         + m1bias_ref[...])
    z = jnp.where(z >= 0.0, z, 0.01 * z)
    o_ref[...] = jnp.dot(z, m2_ref[...], preferred_element_type=jnp.float32) + m2b_ref[...]


def _tc_head(evparts, lw, lb, m1a, m1b_w, m1bias, m2, m2bias):
    return pl.pallas_call(
        _tcd_body,
        out_shape=jax.ShapeDtypeStruct((_P, 128), jnp.float32),
    )(evparts, lw, lb, m1a, m1b_w, m1bias, m2, m2bias)


def _pack_conv_w(W, b, Ws, bs, din_pad):
    din = W.shape[2]
    wt = jnp.transpose(W, (2, 0, 1))
    wt = jnp.pad(wt, ((0, din_pad - din), (0, 0), (0, _HP - _H)))
    wt = wt.reshape(din_pad, _F * _HP)
    ws = jnp.pad(jnp.transpose(Ws), ((0, din_pad - din), (0, _HP - _H)))
    wall = jnp.concatenate([wt, ws], axis=1)
    bt = jnp.pad(b, ((0, 0), (0, _HP - _H))).reshape(_F * _HP)
    bsp = jnp.pad(bs, (0, _HP - _H))
    ball = jnp.concatenate([bt, bsp])[None, :]
    return wall, ball


def kernel(x, edge_index, edge_attr, event1, event2, c0_W, c0_b, c0_Ws, c0_bs,
           c1_W, c1_b, c1_Ws, c1_bs, ln_g, ln_b, lin_W, lin_b, mp1_W, mp1_b,
           mp2_W, mp2_b):
    src = edge_index[0]
    dst = edge_index[1]
    attrF = jnp.transpose(edge_attr.reshape(_E // _BE, _BE, _F),
                          (0, 2, 1)).reshape(-1)
    evcat = jnp.concatenate([event1, event2])

    w0, b0 = _pack_conv_w(c0_W, c0_b, c0_Ws, c0_bs, _D)
    w1, b1 = _pack_conv_w(c1_W, c1_b, c1_Ws, c1_bs, _HP)
    gpad = jnp.pad(ln_g, (0, _HP - _H))[None, :]
    bpad = jnp.pad(ln_b, (0, _HP - _H))[None, :]
    lwp = jnp.pad(jnp.transpose(lin_W), ((0, _HP - _H), (0, _HP - _H)))
    lbp = jnp.pad(lin_b, (0, _HP - _H))[None, :]
    m1a = jnp.pad(jnp.transpose(mp1_W[:, :_H]), ((0, _HP - _H), (0, _HP - _H)))
    m1b_w = jnp.pad(jnp.transpose(mp1_W[:, _H:]), ((0, _HP - _H), (0, _HP - _H)))
    m1bias = jnp.pad(mp1_b, (0, _HP - _H))[None, :]
    m2 = jnp.pad(jnp.transpose(mp2_W), ((0, _HP - _H), (0, 128 - _R)))
    m2bias = jnp.pad(mp2_b, (0, 128 - _R))[None, :]

    yhat0, seed0 = _tc_linear(x, w0, b0)
    parts0 = _sc_agg_full(yhat0, src, dst, attrF, seed0)
    yhat1, seed1 = _tc_ln_linear(parts0, gpad, bpad, w1, b1)
    evparts = _sc_agg_ev(yhat1, src, dst, attrF, seed1, evcat)
    zfull = _tc_head(evparts, lwp, lbp, m1a, m1b_w, m1bias, m2, m2bias)
    return zfull[:, :_R]


# speedup vs baseline: 2.5961x; 1.1705x over previous
import jax
import jax.numpy as jnp
from jax import lax
from jax.experimental import pallas as pl
from jax.experimental.pallas import tpu as pltpu
from jax.experimental.pallas import tpu_sc as plsc

_N = 10000
_E = 320000
_D = 128
_H = 50
_HP = 64
_F = 4
_P = 1024
_R = 3

_NC = 2
_NS = 16
_NW = _NC * _NS
_EPW = _E // _NW
_BE = 80
_NBLK = _EPW // _BE
_NP = 10240
_ROWS = _NP // _NS
_HW = 128
_EVT = 2 * _P
_EVB = _EVT // _NS

_ROWBLK = 400


def _edge_compute(attr_v, rows_v, msg_v):
    def grp(g, carry):
        av = [attr_v[pl.ds(f * _BE + g * 16, 16)] for f in range(_F)]
        for j in range(16):
            e = g * 16 + j
            jv = jnp.full((16,), j, dtype=jnp.int32)
            bf = [
                jnp.take_along_axis(av[f], jv, axis=0,
                                    mode="promise_in_bounds")
                for f in range(_F)
            ]
            for cc in range(_HP // 16):
                m = bf[0] * rows_v[e, pl.ds(cc * 16, 16)]
                for f in range(1, _F):
                    m = m + bf[f] * rows_v[e, pl.ds(f * _HP + cc * 16, 16)]
                msg_v[e, pl.ds(cc * 16, 16)] = m
        return carry

    lax.fori_loop(0, _BE // 16, grp, 0)


def _seed_and_agg(yhat, src, dst, attrF, seed,
                  idx_v, didx_v, attr_v, rows_v, msg_v, accum, sem, sem2,
                  c, s):
    wid = c * _NS + s
    stripe = pl.ds(s * _ROWS, _ROWS)

    zv = jnp.zeros((16,), jnp.float32)

    def zmsg(e, carry):
        for cc in range(_HP // 16):
            msg_v[e, pl.ds(_HP + cc * 16, 16)] = zv
        return carry

    lax.fori_loop(0, _BE, zmsg, 0)
    pltpu.sync_copy(seed.at[c, stripe], accum.at[stripe])
    plsc.subcore_barrier()

    def blk(i, carry):
        base = wid * _EPW + i * _BE
        pltpu.sync_copy(src.at[pl.ds(base, _BE)], idx_v)
        gather = pltpu.async_copy(yhat.at[idx_v], rows_v, sem)

        @pl.when(i > 0)
        def _():
            pltpu.make_async_copy(msg_v, accum.at[didx_v], sem2).wait()

        pltpu.sync_copy(dst.at[pl.ds(base, _BE)], didx_v)
        pltpu.sync_copy(attrF.at[pl.ds(base * _F, _F * _BE)], attr_v)
        gather.wait()
        _edge_compute(attr_v, rows_v, msg_v)
        pltpu.async_copy(msg_v, accum.at[didx_v], sem2, add=True)
        return carry

    lax.fori_loop(0, _NBLK, blk, 0)
    pltpu.make_async_copy(msg_v, accum.at[didx_v], sem2).wait()
    plsc.subcore_barrier()


def _sc_agg_full_body(yhat, src, dst, attrF, seed, out,
                      idx_v, didx_v, attr_v, rows_v, msg_v, accum, sem, sem2):
    c = lax.axis_index("c")
    s = lax.axis_index("s")
    _seed_and_agg(yhat, src, dst, attrF, seed,
                  idx_v, didx_v, attr_v, rows_v, msg_v, accum, sem, sem2, c, s)
    stripe = pl.ds(s * _ROWS, _ROWS)
    pltpu.sync_copy(accum.at[stripe], out.at[c, stripe])


def _sc_agg_ev_body(yhat, src, dst, attrF, seed, evcat, out,
                    idx_v, didx_v, attr_v, rows_v, msg_v,
                    evidx_v, evrows_v, accum, sem, sem2):
    c = lax.axis_index("c")
    s = lax.axis_index("s")
    _seed_and_agg(yhat, src, dst, attrF, seed,
                  idx_v, didx_v, attr_v, rows_v, msg_v, accum, sem, sem2, c, s)
    evb = s * _EVB
    pltpu.sync_copy(evcat.at[pl.ds(evb, _EVB)], evidx_v)
    pltpu.async_copy(accum.at[evidx_v], evrows_v, sem).wait()
    pltpu.sync_copy(evrows_v, out.at[c, pl.ds(evb, _EVB)])


def _sc_mesh():
    return plsc.VectorSubcoreMesh(core_axis_name="c", subcore_axis_name="s")


_SC_COMMON_SCRATCH = [
    pltpu.VMEM((_BE,), jnp.int32),
    pltpu.VMEM((_BE,), jnp.int32),
    pltpu.VMEM((_F * _BE,), jnp.float32),
    pltpu.VMEM((_BE, _F * _HP), jnp.float32),
    pltpu.VMEM((_BE, _HW), jnp.float32),
]


def _sc_agg_full(yhat, src, dst, attrF, seed):
    return pl.kernel(
        _sc_agg_full_body,
        out_type=jax.ShapeDtypeStruct((_NC, _NP, _HW), jnp.float32),
        mesh=_sc_mesh(),
        scratch_types=_SC_COMMON_SCRATCH + [
            pltpu.VMEM_SHARED((_NP, _HW), jnp.float32),
            pltpu.SemaphoreType.DMA,
            pltpu.SemaphoreType.DMA,
        ],
    )(yhat, src, dst, attrF, seed)


def _sc_agg_ev(yhat, src, dst, attrF, seed, evcat):
    return pl.kernel(
        _sc_agg_ev_body,
        out_type=jax.ShapeDtypeStruct((_NC, _EVT, _HW), jnp.float32),
        mesh=_sc_mesh(),
        scratch_types=_SC_COMMON_SCRATCH + [
            pltpu.VMEM((_EVB,), jnp.int32),
            pltpu.VMEM((_EVB, _HW), jnp.float32),
            pltpu.VMEM_SHARED((_NP, _HW), jnp.float32),
            pltpu.SemaphoreType.DMA,
            pltpu.SemaphoreType.DMA,
        ],
    )(yhat, src, dst, attrF, seed, evcat)


def _tca_body(x_ref, w_ref, b_ref, yhat_ref, self_ref):
    t = jnp.dot(x_ref[...], w_ref[...], preferred_element_type=jnp.float32)
    t = t + b_ref[...]
    yhat_ref[...] = t[:, : _F * _HP]
    sf = t[:, _F * _HP:]
    self_ref[0] = jnp.concatenate([sf, jnp.zeros_like(sf)], axis=1)
    self_ref[1] = jnp.zeros((sf.shape[0], _HW), jnp.float32)


def _tc_linear(x, w, b):
    din = x.shape[1]
    return pl.pallas_call(
        _tca_body,
        grid=(_N // _ROWBLK,),
        in_specs=[
            pl.BlockSpec((_ROWBLK, din), lambda i: (i, 0)),
            pl.BlockSpec(w.shape, lambda i: (0, 0)),
            pl.BlockSpec(b.shape, lambda i: (0, 0)),
        ],
        out_specs=[
            pl.BlockSpec((_ROWBLK, _F * _HP), lambda i: (i, 0)),
            pl.BlockSpec((_NC, _ROWBLK, _HW), lambda i: (0, i, 0)),
        ],
        out_shape=[
            jax.ShapeDtypeStruct((_N, _F * _HP), jnp.float32),
            jax.ShapeDtypeStruct((_NC, _NP, _HW), jnp.float32),
        ],
    )(x, w, b)


def _tcb_body(p0_ref, p1_ref, g_ref, bb_ref, w_ref, b_ref,
              yhat_ref, self_ref):
    t = (p0_ref[0] + p1_ref[0])[:, :_HP]
    mean = jnp.sum(t, axis=-1, keepdims=True) * (1.0 / _H)
    d = t - mean
    lane = lax.broadcasted_iota(jnp.int32, t.shape, 1)
    d = jnp.where(lane < _H, d, 0.0)
    var = jnp.sum(d * d, axis=-1, keepdims=True) * (1.0 / _H)
    h1 = d * lax.rsqrt(var + 1e-5) * g_ref[...] + bb_ref[...]
    t2 = jnp.dot(h1, w_ref[...], preferred_element_type=jnp.float32)
    t2 = t2 + b_ref[...]
    yhat_ref[...] = t2[:, : _F * _HP]
    sf = t2[:, _F * _HP:]
    self_ref[0] = jnp.concatenate([sf, jnp.zeros_like(sf)], axis=1)
    self_ref[1] = jnp.zeros((sf.shape[0], _HW), jnp.float32)


def _tc_ln_linear(parts, g, bb, w, b):
    return pl.pallas_call(
        _tcb_body,
        grid=(_N // _ROWBLK,),
        in_specs=[
            pl.BlockSpec((1, _ROWBLK, _HW), lambda i: (0, i, 0)),
            pl.BlockSpec((1, _ROWBLK, _HW), lambda i: (1, i, 0)),
            pl.BlockSpec(g.shape, lambda i: (0, 0)),
            pl.BlockSpec(bb.shape, lambda i: (0, 0)),
            pl.BlockSpec(w.shape, lambda i: (0, 0)),
            pl.BlockSpec(b.shape, lambda i: (0, 0)),
        ],
        out_specs=[
            pl.BlockSpec((_ROWBLK, _F * _HP), lambda i: (i, 0)),
            pl.BlockSpec((_NC, _ROWBLK, _HW), lambda i: (0, i, 0)),
        ],
        out_shape=[
            jax.ShapeDtypeStruct((_N, _F * _HP), jnp.float32),
            jax.ShapeDtypeStruct((_NC, _NP, _HW), jnp.float32),
        ],
    )(parts, parts, g, bb, w, b)


def _tcd_body(ev_ref, lw_ref, lb_ref, m1a_ref, m1b_ref, m1bias_ref,
              m2_ref, m2b_ref, o_ref):
    ht = (ev_ref[0] + ev_ref[1])[:, :_HP]
    h2 = jnp.dot(ht, lw_ref[...], preferred_element_type=jnp.float32)
    h2 = h2 + lb_ref[...]
    ea = h2[:_P]
    eb = h2[_P:]
    z = (jnp.dot(ea, m1a_ref[...], preferred_element_type=jnp.float32)
         + jnp.dot(eb, m1b_ref[...], preferred_element_type=jnp.float32)
         + m1bias_ref[...])
    z = jnp.where(z >= 0.0, z, 0.01 * z)
    o_ref[...] = jnp.dot(z, m2_ref[...], preferred_element_type=jnp.float32) + m2b_ref[...]


def _tc_head(evparts, lw, lb, m1a, m1b_w, m1bias, m2, m2bias):
    return pl.pallas_call(
        _tcd_body,
        out_shape=jax.ShapeDtypeStruct((_P, 128), jnp.float32),
    )(evparts, lw, lb, m1a, m1b_w, m1bias, m2, m2bias)


def _pack_conv_w(W, b, Ws, bs, din_pad):
    din = W.shape[2]
    wt = jnp.transpose(W, (2, 0, 1))
    wt = jnp.pad(wt, ((0, din_pad - din), (0, 0), (0, _HP - _H)))
    wt = wt.reshape(din_pad, _F * _HP)
    ws = jnp.pad(jnp.transpose(Ws), ((0, din_pad - din), (0, _HP - _H)))
    wall = jnp.concatenate([wt, ws], axis=1)
    bt = jnp.pad(b, ((0, 0), (0, _HP - _H))).reshape(_F * _HP)
    bsp = jnp.pad(bs, (0, _HP - _H))
    ball = jnp.concatenate([bt, bsp])[None, :]
    return wall, ball


def kernel(x, edge_index, edge_attr, event1, event2, c0_W, c0_b, c0_Ws, c0_bs,
           c1_W, c1_b, c1_Ws, c1_bs, ln_g, ln_b, lin_W, lin_b, mp1_W, mp1_b,
           mp2_W, mp2_b):
    src = edge_index[0]
    dst = edge_index[1]
    attrF = jnp.transpose(edge_attr.reshape(_E // _BE, _BE, _F),
                          (0, 2, 1)).reshape(-1)
    evcat = jnp.concatenate([event1, event2])

    w0, b0 = _pack_conv_w(c0_W, c0_b, c0_Ws, c0_bs, _D)
    w1, b1 = _pack_conv_w(c1_W, c1_b, c1_Ws, c1_bs, _HP)
    gpad = jnp.pad(ln_g, (0, _HP - _H))[None, :]
    bpad = jnp.pad(ln_b, (0, _HP - _H))[None, :]
    lwp = jnp.pad(jnp.transpose(lin_W), ((0, _HP - _H), (0, _HP - _H)))
    lbp = jnp.pad(lin_b, (0, _HP - _H))[None, :]
    m1a = jnp.pad(jnp.transpose(mp1_W[:, :_H]), ((0, _HP - _H), (0, _HP - _H)))
    m1b_w = jnp.pad(jnp.transpose(mp1_W[:, _H:]), ((0, _HP - _H), (0, _HP - _H)))
    m1bias = jnp.pad(mp1_b, (0, _HP - _H))[None, :]
    m2 = jnp.pad(jnp.transpose(mp2_W), ((0, _HP - _H), (0, 128 - _R)))
    m2bias = jnp.pad(mp2_b, (0, 128 - _R))[None, :]

    yhat0, seed0 = _tc_linear(x, w0, b0)
    parts0 = _sc_agg_full(yhat0, src, dst, attrF, seed0)
    yhat1, seed1 = _tc_ln_linear(parts0, gpad, bpad, w1, b1)
    evparts = _sc_agg_ev(yhat1, src, dst, attrF, seed1, evcat)
    zfull = _tc_head(evparts, lwp, lbp, m1a, m1b_w, m1bias, m2, m2bias)
    return zfull[:, :_R]
